# Initial kernel scaffold; baseline (speedup 1.0000x reference)
import jax
import jax.numpy as jnp
from jax.experimental import pallas as pl


def kernel(x, src_mask, pitch_target, energy_target, duration_target, max_len, pitch_bins, energy_bins, pitch_emb, energy_emb, dp, pp, ep):
    raise NotImplementedError("write your pallas kernel here")


# speedup vs baseline: 25.9598x; 25.9598x over previous
import functools

import jax
import jax.numpy as jnp
from jax import lax
from jax.experimental import pallas as pl
from jax.experimental.pallas import tpu as pltpu
from jax.experimental.pallas import tpu_sc as plsc

B, L, D = 16, 512, 256
NBINS = 256
MAXLEN = 2048
LP = L + 1


def _vp(xin, k, w1s_ref, b1s_ref, g1s_ref, be1s_ref,
        w2s_ref, b2s_ref, g2s_ref, be2s_ref, lws_ref, lbs_ref, keep_col):
    def conv_ln(h, w_ref, b_ref, g_ref, be_ref):
        hprev = jnp.concatenate([jnp.zeros((1, h.shape[1]), jnp.float32), h[:-1]], axis=0)
        hnext = jnp.concatenate([h[1:], jnp.zeros((1, h.shape[1]), jnp.float32)], axis=0)
        cat = jnp.concatenate([hprev, h, hnext], axis=1)
        y = jnp.dot(cat, w_ref[k], preferred_element_type=jnp.float32) + b_ref[k]
        y = jnp.maximum(y, 0.0)
        m = jnp.mean(y, axis=1, keepdims=True)
        v = jnp.mean((y - m) ** 2, axis=1, keepdims=True)
        return (y - m) * jax.lax.rsqrt(v + 1e-5) * g_ref[k] + be_ref[k]

    h = conv_ln(xin, w1s_ref, b1s_ref, g1s_ref, be1s_ref)
    h = conv_ln(h, w2s_ref, b2s_ref, g2s_ref, be2s_ref)
    out = jnp.sum(h * lws_ref[k], axis=1, keepdims=True) + lbs_ref[k]
    return out * keep_col


def _tc_body(x_ref, pt_ref, et_ref, dur_ref, keep_ref,
             pbins_ref, ebins_ref, pemb_ref, eemb_ref,
             w1s_ref, b1s_ref, g1s_ref, be1s_ref,
             w2s_ref, b2s_ref, g2s_ref, be2s_ref, lws_ref, lbs_ref,
             x2_ref, ldp_ref, pp_ref, ep_ref, fidx_ref, mmask_ref):
    b = pl.program_id(0)
    x = x_ref[0]
    keep_col = keep_ref[0]

    lane_iota = lax.broadcasted_iota(jnp.int32, (1, NBINS), 1)

    def embed(t_col, bins_ref, emb_ref):
        cnt = jnp.sum((bins_ref[0][None, :] < t_col).astype(jnp.int32),
                      axis=1, keepdims=True)
        oh = (cnt == lane_iota).astype(jnp.float32)
        return jnp.dot(oh, emb_ref[...], preferred_element_type=jnp.float32)

    args = (w1s_ref, b1s_ref, g1s_ref, be1s_ref,
            w2s_ref, b2s_ref, g2s_ref, be2s_ref, lws_ref, lbs_ref, keep_col)

    ldp_ref[0] = _vp(x, 0, *args)
    pp_ref[0] = _vp(x, 1, *args)

    x1 = x + embed(pt_ref[0], pbins_ref, pemb_ref)
    ep_ref[0] = _vp(x1, 2, *args)

    x2 = x1 + embed(et_ref[0], ebins_ref, eemb_ref)
    x2_ref[0, pl.ds(0, L), :] = x2
    x2_ref[0, pl.ds(L, 1), :] = jnp.zeros((1, D), jnp.float32)

    dur_col = dur_ref[0].astype(jnp.float32)
    r = lax.broadcasted_iota(jnp.int32, (L, L), 0)
    c = lax.broadcasted_iota(jnp.int32, (L, L), 1)
    tri = (c <= r).astype(jnp.float32)
    cum_col = jnp.dot(tri, dur_col, preferred_element_type=jnp.float32)
    t_row = lax.broadcasted_iota(jnp.int32, (1, MAXLEN), 1).astype(jnp.float32)
    idx_row = jnp.sum((cum_col <= t_row).astype(jnp.int32), axis=0, keepdims=True)
    idx_row = jnp.minimum(idx_row, L - 1)
    mel_total = jnp.sum(dur_col)
    mel_len = jnp.minimum(mel_total, float(MAXLEN))
    mask_row = (t_row >= mel_len)
    fidx_ref[0] = jnp.where(mask_row, b * LP + L, b * LP + idx_row)
    mmask_ref[0] = mask_row.astype(jnp.int32)


def _tc_stage(x, pt_col, et_col, dur_col, keep_col, pbins, ebins,
              pitch_emb, energy_emb, w1s, b1s, g1s, be1s,
              w2s, b2s, g2s, be2s, lws, lbs):
    full3 = lambda s: pl.BlockSpec(s, lambda b: (0, 0, 0))
    full2 = lambda s: pl.BlockSpec(s, lambda b: (0, 0))
    per_b3 = lambda s: pl.BlockSpec(s, lambda b: (b, 0, 0))
    return pl.pallas_call(
        _tc_body,
        grid=(B,),
        in_specs=[
            per_b3((1, L, D)),
            per_b3((1, L, 1)),
            per_b3((1, L, 1)),
            per_b3((1, L, 1)),
            per_b3((1, L, 1)),
            full2((1, NBINS)),
            full2((1, NBINS)),
            full2((NBINS, D)),
            full2((NBINS, D)),
            full3((3, 3 * D, D)),
            full3((3, 1, D)),
            full3((3, 1, D)),
            full3((3, 1, D)),
            full3((3, 3 * D, D)),
            full3((3, 1, D)),
            full3((3, 1, D)),
            full3((3, 1, D)),
            full3((3, 1, D)),
            full3((3, 1, 1)),
        ],
        out_specs=[
            per_b3((1, LP, D)),
            per_b3((1, L, 1)),
            per_b3((1, L, 1)),
            per_b3((1, L, 1)),
            per_b3((1, 1, MAXLEN)),
            per_b3((1, 1, MAXLEN)),
        ],
        out_shape=[
            jax.ShapeDtypeStruct((B, LP, D), jnp.float32),
            jax.ShapeDtypeStruct((B, L, 1), jnp.float32),
            jax.ShapeDtypeStruct((B, L, 1), jnp.float32),
            jax.ShapeDtypeStruct((B, L, 1), jnp.float32),
            jax.ShapeDtypeStruct((B, 1, MAXLEN), jnp.int32),
            jax.ShapeDtypeStruct((B, 1, MAXLEN), jnp.int32),
        ],
    )(x, pt_col, et_col, dur_col, keep_col, pbins, ebins, pitch_emb,
      energy_emb, w1s, b1s, g1s, be1s, w2s, b2s, g2s, be2s, lws, lbs)


def _sc_gather(table, flat_idx):
    info = plsc.get_sparse_core_info()
    NC, NS = info.num_cores, info.num_subcores
    NW = NC * NS
    ROWS = B * MAXLEN
    b_per_w = ROWS // NW
    CH = 128
    n_ch = b_per_w // CH

    @functools.partial(
        pl.kernel,
        mesh=plsc.VectorSubcoreMesh(core_axis_name="c", subcore_axis_name="s"),
        out_type=jax.ShapeDtypeStruct((ROWS, D), jnp.float32),
        scratch_types=[
            pltpu.VMEM((CH,), jnp.int32),
            pltpu.VMEM((CH, D), jnp.float32),
            pltpu.SemaphoreType.DMA,
        ],
    )
    def k(table_hbm, idx_hbm, out_hbm, idx_v, rows_v, sem):
        wid = lax.axis_index("s") * NC + lax.axis_index("c")
        base = wid * b_per_w
        for c in range(n_ch):
            off = base + c * CH
            pltpu.sync_copy(idx_hbm.at[pl.ds(off, CH)], idx_v)
            pltpu.async_copy(table_hbm.at[idx_v], rows_v, sem).wait()
            pltpu.sync_copy(rows_v, out_hbm.at[pl.ds(off, CH)])

    return k(table, flat_idx)


def kernel(x, src_mask, pitch_target, energy_target, duration_target, max_len,
           pitch_bins, energy_bins, pitch_emb, energy_emb, dp, pp, ep):
    def pack(params):
        w1, b1, g1, be1, w2, b2, g2, be2, lw, lb = params
        return (w1.reshape(3 * D, D), b1, g1, be1,
                w2.reshape(3 * D, D), b2, g2, be2, lw[:, 0], lb)

    packed = [pack(p) for p in (dp, pp, ep)]
    stack = lambda i: jnp.stack([p[i] for p in packed])
    w1s = stack(0)
    b1s = stack(1)[:, None, :]
    g1s = stack(2)[:, None, :]
    be1s = stack(3)[:, None, :]
    w2s = stack(4)
    b2s = stack(5)[:, None, :]
    g2s = stack(6)[:, None, :]
    be2s = stack(7)[:, None, :]
    lws = stack(8)[:, None, :]
    lbs = stack(9)[:, None, :]

    pbins = jnp.concatenate([pitch_bins, jnp.full((1,), 2.0, jnp.float32)])[None, :]
    ebins = jnp.concatenate([energy_bins, jnp.full((1,), 2.0, jnp.float32)])[None, :]
    pt_col = pitch_target[:, :, None]
    et_col = energy_target[:, :, None]
    dur_col = duration_target[:, :, None]
    keep_col = 1.0 - src_mask[:, :, None].astype(jnp.float32)

    x2t, ldp, ppred, epred, fidx, mmask = _tc_stage(
        x, pt_col, et_col, dur_col, keep_col, pbins, ebins,
        pitch_emb, energy_emb, w1s, b1s, g1s, be1s, w2s, b2s, g2s, be2s,
        lws, lbs)

    out = _sc_gather(x2t.reshape(B * LP, D), fidx.reshape(B * MAXLEN))
    out = out.reshape(B, MAXLEN, D)

    log_duration_prediction = ldp.reshape(B, L)
    pitch_prediction = ppred.reshape(B, L)
    energy_prediction = epred.reshape(B, L)
    mel_len = jnp.minimum(jnp.sum(duration_target, axis=1), max_len)
    mel_mask = mmask.reshape(B, MAXLEN).astype(bool)
    return (out, pitch_prediction, energy_prediction, log_duration_prediction,
            duration_target, mel_len, mel_mask)


# speedup vs baseline: 33.5348x; 1.2918x over previous
import functools

import jax
import jax.numpy as jnp
from jax import lax
from jax.experimental import pallas as pl
from jax.experimental.pallas import tpu as pltpu
from jax.experimental.pallas import tpu_sc as plsc

B, L, D = 16, 512, 256
NBINS = 256
MAXLEN = 2048
LP = L + 1


def _vp(xin, k, w1s_ref, b1s_ref, g1s_ref, be1s_ref,
        w2s_ref, b2s_ref, g2s_ref, be2s_ref, lws_ref, lbs_ref, keep_col):
    def conv_ln(h, w_ref, b_ref, g_ref, be_ref):
        hprev = jnp.concatenate([jnp.zeros((1, h.shape[1]), jnp.float32), h[:-1]], axis=0)
        hnext = jnp.concatenate([h[1:], jnp.zeros((1, h.shape[1]), jnp.float32)], axis=0)
        cat = jnp.concatenate([hprev, h, hnext], axis=1)
        y = jnp.dot(cat, w_ref[k], preferred_element_type=jnp.float32) + b_ref[k]
        y = jnp.maximum(y, 0.0)
        m = jnp.mean(y, axis=1, keepdims=True)
        v = jnp.mean((y - m) ** 2, axis=1, keepdims=True)
        return (y - m) * jax.lax.rsqrt(v + 1e-5) * g_ref[k] + be_ref[k]

    h = conv_ln(xin, w1s_ref, b1s_ref, g1s_ref, be1s_ref)
    h = conv_ln(h, w2s_ref, b2s_ref, g2s_ref, be2s_ref)
    out = jnp.sum(h * lws_ref[k], axis=1, keepdims=True) + lbs_ref[k]
    return out * keep_col


def _embed_body(x_ref, pt_ref, et_ref, dur_ref,
                pbins_ref, ebins_ref, pemb_ref, eemb_ref,
                x1_ref, x2_ref, fidx_ref, mmask_ref):
    b = pl.program_id(0)
    x = x_ref[0]

    lane_iota = lax.broadcasted_iota(jnp.int32, (1, NBINS), 1)

    def embed(t_col, bins_ref, emb_ref):
        cnt = jnp.sum((bins_ref[0][None, :] < t_col).astype(jnp.int32),
                      axis=1, keepdims=True)
        oh = (cnt == lane_iota).astype(jnp.float32)
        return jnp.dot(oh, emb_ref[...], preferred_element_type=jnp.float32)

    x1 = x + embed(pt_ref[0], pbins_ref, pemb_ref)
    x1_ref[0] = x1
    x2 = x1 + embed(et_ref[0], ebins_ref, eemb_ref)
    x2_ref[0, pl.ds(0, L), :] = x2
    x2_ref[0, pl.ds(L, 1), :] = jnp.zeros((1, D), jnp.float32)

    dur_col = dur_ref[0].astype(jnp.float32)
    r = lax.broadcasted_iota(jnp.int32, (L, L), 0)
    c = lax.broadcasted_iota(jnp.int32, (L, L), 1)
    tri = (c <= r).astype(jnp.float32)
    cum_col = jnp.dot(tri, dur_col, preferred_element_type=jnp.float32)
    t_row = lax.broadcasted_iota(jnp.int32, (1, MAXLEN), 1).astype(jnp.float32)
    idx_row = jnp.sum((cum_col <= t_row).astype(jnp.int32), axis=0, keepdims=True)
    idx_row = jnp.minimum(idx_row, L - 1)
    mel_total = jnp.sum(dur_col)
    mel_len = jnp.minimum(mel_total, float(MAXLEN))
    mask_row = (t_row >= mel_len)
    fidx_ref[0] = jnp.where(mask_row, b * LP + L, b * LP + idx_row)
    mmask_ref[0] = mask_row.astype(jnp.int32)


def _embed_stage(x, pt_col, et_col, dur_col, pbins, ebins,
                 pitch_emb, energy_emb):
    full2 = lambda s: pl.BlockSpec(s, lambda b: (0, 0))
    per_b3 = lambda s: pl.BlockSpec(s, lambda b: (b, 0, 0))
    return pl.pallas_call(
        _embed_body,
        grid=(B,),
        in_specs=[
            per_b3((1, L, D)),
            per_b3((1, L, 1)),
            per_b3((1, L, 1)),
            per_b3((1, L, 1)),
            full2((1, NBINS)),
            full2((1, NBINS)),
            full2((NBINS, D)),
            full2((NBINS, D)),
        ],
        out_specs=[
            per_b3((1, L, D)),
            per_b3((1, LP, D)),
            per_b3((1, 1, MAXLEN)),
            per_b3((1, 1, MAXLEN)),
        ],
        out_shape=[
            jax.ShapeDtypeStruct((B, L, D), jnp.float32),
            jax.ShapeDtypeStruct((B, LP, D), jnp.float32),
            jax.ShapeDtypeStruct((B, 1, MAXLEN), jnp.int32),
            jax.ShapeDtypeStruct((B, 1, MAXLEN), jnp.int32),
        ],
    )(x, pt_col, et_col, dur_col, pbins, ebins, pitch_emb, energy_emb)


def _pred_body(x_ref, x1_ref, keep_ref,
               w1s_ref, b1s_ref, g1s_ref, be1s_ref,
               w2s_ref, b2s_ref, g2s_ref, be2s_ref, lws_ref, lbs_ref,
               ldp_ref, pp_ref, ep_ref):
    x = x_ref[0]
    keep_col = keep_ref[0]
    args = (w1s_ref, b1s_ref, g1s_ref, be1s_ref,
            w2s_ref, b2s_ref, g2s_ref, be2s_ref, lws_ref, lbs_ref, keep_col)
    ldp_ref[0] = _vp(x, 0, *args)
    pp_ref[0] = _vp(x, 1, *args)
    ep_ref[0] = _vp(x1_ref[0], 2, *args)


def _pred_stage(x, x1, keep_col, w1s, b1s, g1s, be1s,
                w2s, b2s, g2s, be2s, lws, lbs):
    full3 = lambda s: pl.BlockSpec(s, lambda b: (0, 0, 0))
    per_b3 = lambda s: pl.BlockSpec(s, lambda b: (b, 0, 0))
    return pl.pallas_call(
        _pred_body,
        grid=(B,),
        in_specs=[
            per_b3((1, L, D)),
            per_b3((1, L, D)),
            per_b3((1, L, 1)),
            full3((3, 3 * D, D)),
            full3((3, 1, D)),
            full3((3, 1, D)),
            full3((3, 1, D)),
            full3((3, 3 * D, D)),
            full3((3, 1, D)),
            full3((3, 1, D)),
            full3((3, 1, D)),
            full3((3, 1, D)),
            full3((3, 1, 1)),
        ],
        out_specs=[
            per_b3((1, L, 1)),
            per_b3((1, L, 1)),
            per_b3((1, L, 1)),
        ],
        out_shape=[
            jax.ShapeDtypeStruct((B, L, 1), jnp.float32),
            jax.ShapeDtypeStruct((B, L, 1), jnp.float32),
            jax.ShapeDtypeStruct((B, L, 1), jnp.float32),
        ],
    )(x, x1, keep_col, w1s, b1s, g1s, be1s, w2s, b2s, g2s, be2s, lws, lbs)


def _sc_gather(table, flat_idx):
    info = plsc.get_sparse_core_info()
    NC, NS = info.num_cores, info.num_subcores
    NW = NC * NS
    ROWS = B * MAXLEN
    b_per_w = ROWS // NW
    CH = 128
    n_ch = b_per_w // CH

    @functools.partial(
        pl.kernel,
        mesh=plsc.VectorSubcoreMesh(core_axis_name="c", subcore_axis_name="s"),
        out_type=jax.ShapeDtypeStruct((ROWS, D), jnp.float32),
        scratch_types=[
            pltpu.VMEM((b_per_w,), jnp.int32),
            pltpu.VMEM((CH, D), jnp.float32),
            pltpu.VMEM((CH, D), jnp.float32),
            pltpu.SemaphoreType.DMA,
            pltpu.SemaphoreType.DMA,
            pltpu.SemaphoreType.DMA,
            pltpu.SemaphoreType.DMA,
        ],
    )
    def k(table_hbm, idx_hbm, out_hbm, idx_v, rows0, rows1,
          gsem0, gsem1, ssem0, ssem1):
        wid = lax.axis_index("s") * NC + lax.axis_index("c")
        base = wid * b_per_w
        pltpu.sync_copy(idx_hbm.at[pl.ds(base, b_per_w)], idx_v)
        rows = (rows0, rows1)
        gsems = (gsem0, gsem1)
        ssems = (ssem0, ssem1)

        def gather(c):
            return pltpu.async_copy(
                table_hbm.at[idx_v.at[pl.ds(c * CH, CH)]],
                rows[c % 2], gsems[c % 2])

        def store(c):
            return pltpu.async_copy(
                rows[c % 2], out_hbm.at[pl.ds(base + c * CH, CH)],
                ssems[c % 2])

        stores = [None, None]
        g = gather(0)
        for c in range(n_ch):
            j = (c + 1) % 2
            if c + 1 < n_ch:
                if stores[j] is not None:
                    stores[j].wait()
                g_next = gather(c + 1)
            g.wait()
            stores[c % 2] = store(c)
            if c + 1 < n_ch:
                g = g_next
        stores[0].wait()
        stores[1].wait()

    return k(table, flat_idx)


def kernel(x, src_mask, pitch_target, energy_target, duration_target, max_len,
           pitch_bins, energy_bins, pitch_emb, energy_emb, dp, pp, ep):
    def pack(params):
        w1, b1, g1, be1, w2, b2, g2, be2, lw, lb = params
        return (w1.reshape(3 * D, D), b1, g1, be1,
                w2.reshape(3 * D, D), b2, g2, be2, lw[:, 0], lb)

    packed = [pack(p) for p in (dp, pp, ep)]
    stack = lambda i: jnp.stack([p[i] for p in packed])
    w1s = stack(0)
    b1s = stack(1)[:, None, :]
    g1s = stack(2)[:, None, :]
    be1s = stack(3)[:, None, :]
    w2s = stack(4)
    b2s = stack(5)[:, None, :]
    g2s = stack(6)[:, None, :]
    be2s = stack(7)[:, None, :]
    lws = stack(8)[:, None, :]
    lbs = stack(9)[:, None, :]

    pbins = jnp.concatenate([pitch_bins, jnp.full((1,), 2.0, jnp.float32)])[None, :]
    ebins = jnp.concatenate([energy_bins, jnp.full((1,), 2.0, jnp.float32)])[None, :]
    pt_col = pitch_target[:, :, None]
    et_col = energy_target[:, :, None]
    dur_col = duration_target[:, :, None]
    keep_col = 1.0 - src_mask[:, :, None].astype(jnp.float32)

    x1, x2t, fidx, mmask = _embed_stage(
        x, pt_col, et_col, dur_col, pbins, ebins, pitch_emb, energy_emb)

    out = _sc_gather(x2t.reshape(B * LP, D), fidx.reshape(B * MAXLEN))
    out = out.reshape(B, MAXLEN, D)

    ldp, ppred, epred = _pred_stage(
        x, x1, keep_col, w1s, b1s, g1s, be1s, w2s, b2s, g2s, be2s, lws, lbs)

    log_duration_prediction = ldp.reshape(B, L)
    pitch_prediction = ppred.reshape(B, L)
    energy_prediction = epred.reshape(B, L)
    mel_len = jnp.minimum(jnp.sum(duration_target, axis=1), max_len)
    mel_mask = mmask.reshape(B, MAXLEN).astype(bool)
    return (out, pitch_prediction, energy_prediction, log_duration_prediction,
            duration_target, mel_len, mel_mask)
